# baseline (speedup 1.0000x reference)
import jax
import jax.numpy as jnp
from jax import lax
from jax.experimental import pallas as pl
from jax.experimental.pallas import tpu as pltpu
from jax.experimental.pallas import tpu_sc as plsc

VOC_SIZE = 1000000
EMB_SIZE = 64
ROW_PAD = 128
BATCH = 16384

NUM_CORES = 2
NUM_SUBCORES = 16
NUM_WORKERS = NUM_CORES * NUM_SUBCORES
BPW = BATCH // NUM_WORKERS
CHUNK = 128
NCHUNK = BPW // CHUNK
LANES = 16
NGROUP_C = CHUNK // LANES


def _rsqrt(v):
    xi = plsc.bitcast(v, jnp.int32)
    yi = jnp.full((LANES,), 0x5F3759DF, jnp.int32) - lax.shift_right_logical(
        xi, jnp.full((LANES,), 1, jnp.int32))
    y = plsc.bitcast(yi, jnp.float32)
    h = v * 0.5
    for _ in range(3):
        y = y * (1.5 - h * y * y)
    return y


def _body(idx0_hbm, idx1_hbm, off0_hbm, off1_hbm, table_hbm, out_hbm,
          idx0_v, idx1_v, off0_v, off1_v, rows0_v, rows1_v, out_v,
          sem0, sem1):
    c = lax.axis_index("c")
    s = lax.axis_index("s")
    wid = s * NUM_CORES + c
    base = wid * BPW

    iota = lax.iota(jnp.int32, LANES)
    zero = jnp.zeros((LANES,), jnp.float32)

    def chunk(h, carry):
        hbase = base + h * CHUNK
        pltpu.sync_copy(idx0_hbm.at[pl.ds(hbase, CHUNK)], idx0_v)
        pltpu.sync_copy(idx1_hbm.at[pl.ds(hbase, CHUNK)], idx1_v)
        pltpu.sync_copy(off0_hbm.at[pl.ds(hbase, CHUNK)], off0_v)
        pltpu.sync_copy(off1_hbm.at[pl.ds(hbase, CHUNK)], off1_v)
        cp0 = pltpu.async_copy(table_hbm.at[idx0_v], rows0_v, sem0)
        cp1 = pltpu.async_copy(table_hbm.at[idx1_v], rows1_v, sem1)
        cp0.wait()
        cp1.wait()

        def group(g, carry2):
            row_idx = g * LANES + iota
            a_off = off0_v[pl.ds(g * LANES, LANES)]
            b_off = off1_v[pl.ds(g * LANES, LANES)]
            dot, n0, n1 = zero, zero, zero
            for j in range(EMB_SIZE):
                cj = jnp.full((LANES,), j, jnp.int32)
                a = plsc.load_gather(rows0_v, [row_idx, cj + a_off])
                b = plsc.load_gather(rows1_v, [row_idx, cj + b_off])
                dot = dot + a * b
                n0 = n0 + a * a
                n1 = n1 + b * b
            out_v[pl.ds(h * CHUNK + g * LANES, LANES)] = dot * _rsqrt(n0 * n1)
            return carry2

        lax.fori_loop(0, NGROUP_C, group, 0)
        return carry

    lax.fori_loop(0, NCHUNK, chunk, 0)
    pltpu.sync_copy(out_v, out_hbm.at[pl.ds(base, BPW)])


@jax.jit
def _w2vec_sc(idx0, idx1, off0, off1, table2):
    mesh = plsc.VectorSubcoreMesh(core_axis_name="c", subcore_axis_name="s")
    return pl.kernel(
        _body,
        mesh=mesh,
        out_type=jax.ShapeDtypeStruct((BATCH,), jnp.float32),
        scratch_types=[
            pltpu.VMEM((CHUNK,), jnp.int32),
            pltpu.VMEM((CHUNK,), jnp.int32),
            pltpu.VMEM((CHUNK,), jnp.int32),
            pltpu.VMEM((CHUNK,), jnp.int32),
            pltpu.VMEM((CHUNK, ROW_PAD), jnp.float32),
            pltpu.VMEM((CHUNK, ROW_PAD), jnp.float32),
            pltpu.VMEM((BPW,), jnp.float32),
            pltpu.SemaphoreType.DMA,
            pltpu.SemaphoreType.DMA,
        ],
        compiler_params=pltpu.CompilerParams(
            needs_layout_passes=False, use_tc_tiling_on_sc=True),
    )(idx0, idx1, off0, off1, table2)


def kernel(x, table):
    table2 = table.reshape(VOC_SIZE // 2, ROW_PAD)
    idx = lax.shift_right_logical(x, 1)
    off = (x & 1) * EMB_SIZE
    return _w2vec_sc(idx[0], idx[1], off[0], off[1], table2)
